# baseline (speedup 1.0000x reference)
import dataclasses
import functools

import jax
import jax.numpy as jnp
from jax import lax
from jax.experimental import pallas as pl
from jax.experimental.pallas import tpu as pltpu
from jax.experimental.pallas import tpu_sc as plsc

_B, _N1, _N2 = 16, 1024, 1024
_R = 256
_NBLK = _N1 // _R
_NROWS = _B * _N1
_GW = 128
_NC, _NS = 2, 16
_NW = _NC * _NS
_BPW = _NROWS // _NW


def _gt_idx_body(tgt_ref, gt_ref, idx16_ref, lane_ref):
    b = pl.program_id(0)
    r = pl.program_id(1)
    nt = tgt_ref[b]
    gx = gt_ref[0]
    col = lax.broadcasted_iota(jnp.int32, (_R, _N2), 1)
    neg = jnp.float32(-jnp.inf)
    mg = jnp.where(col < nt, gx, neg)
    g1 = jnp.max(mg, axis=1, keepdims=True)
    gidx = jnp.min(jnp.where(mg == g1, col, _N2), axis=1, keepdims=True)
    row = b * _N1 + r * _R + lax.broadcasted_iota(jnp.int32, (_R, 1), 0)
    idx16_ref[...] = row * (_N2 // _GW) + lax.shift_right_logical(gidx, 7)
    lane_ref[...] = lax.bitwise_and(gidx, _GW - 1)


def _top2_body(tgt_ref, pred_ref, m1_ref, m2_ref):
    b = pl.program_id(0)
    nt = tgt_ref[b]
    px = pred_ref[0]
    col = lax.broadcasted_iota(jnp.int32, (_R, _N2), 1)
    neg = jnp.float32(-jnp.inf)
    mp = jnp.where(col < nt, px, neg)
    h = jnp.maximum(mp[:, :512], mp[:, 512:])
    l = jnp.minimum(mp[:, :512], mp[:, 512:])
    for w in (256, 128):
        h1, h2 = h[:, :w], h[:, w:]
        l = jnp.maximum(jnp.minimum(h1, h2), jnp.maximum(l[:, :w], l[:, w:]))
        h = jnp.maximum(h1, h2)
    m1 = jnp.max(h, axis=1, keepdims=True)
    is_m1 = h == m1
    m2h_strict = jnp.max(jnp.where(is_m1, neg, h), axis=1, keepdims=True)
    dup = jnp.sum(jnp.where(is_m1, 1, 0), axis=1, keepdims=True) > 1
    m2h = jnp.where(dup, m1, m2h_strict)
    m1_ref[...] = m1
    m2_ref[...] = jnp.maximum(m2h, jnp.max(l, axis=1, keepdims=True))


def _sc_gather(table, idx16, lane):
    mesh = plsc.VectorSubcoreMesh(core_axis_name="c", subcore_axis_name="s")
    cp = pltpu.CompilerParams()
    if "needs_layout_passes" in pltpu.CompilerParams.__dataclass_fields__:
        cp = dataclasses.replace(cp, needs_layout_passes=False)

    @functools.partial(
        pl.kernel,
        mesh=mesh,
        compiler_params=cp,
        out_type=jax.ShapeDtypeStruct((_NROWS,), jnp.float32),
        scratch_types=[
            pltpu.VMEM((_BPW,), jnp.int32),
            pltpu.VMEM((_BPW,), jnp.int32),
            pltpu.VMEM((_BPW, _GW), jnp.float32),
            pltpu.VMEM((_BPW,), jnp.float32),
            pltpu.SemaphoreType.DMA,
        ],
    )
    def k(table_hbm, idx_hbm, lane_hbm, out_hbm, idx_v, lane_v, rows_v, out_v, sem):
        wid = lax.axis_index("s") * _NC + lax.axis_index("c")
        base = wid * _BPW
        pltpu.sync_copy(idx_hbm.at[pl.ds(base, _BPW)], idx_v)
        pltpu.sync_copy(lane_hbm.at[pl.ds(base, _BPW)], lane_v)
        pltpu.async_copy(table_hbm.at[idx_v], rows_v, sem).wait()
        i16 = lax.iota(jnp.int32, 16)

        @pl.loop(0, _BPW, step=16)
        def _(c):
            rid = i16 + c
            lv = lane_v[pl.ds(c, 16)]
            out_v[pl.ds(c, 16)] = plsc.load_gather(rows_v, [rid, lv])

        pltpu.sync_copy(out_v, out_hbm.at[pl.ds(base, _BPW)])

    return k(table, idx16, lane)


def _combine_body(src_ref, m1_ref, m2_ref, pag_ref, out_ref):
    m1 = m1_ref[...]
    m2 = m2_ref[...]
    pag = pag_ref[...]
    bi = lax.broadcasted_iota(jnp.int32, (_B, _N1), 0)
    ri = lax.broadcasted_iota(jnp.int32, (_B, _N1), 1)
    ns_b = jnp.zeros((_B, _N1), jnp.int32)
    for b in range(_B):
        ns_b = jnp.where(bi == b, src_ref[b], ns_b)
    sel = jnp.where(pag == m1, m2, m1)
    contrib = jnp.log(sel) - jnp.log(pag)
    total = jnp.sum(jnp.where(ri < ns_b, contrib, 0.0))
    n_sum = lax.fori_loop(
        0, _B, lambda i, s: s + src_ref[i].astype(jnp.float32), jnp.float32(0.0)
    )
    out_ref[0, 0] = total / n_sum


def kernel(pred_dsmat, gt_perm, src_ns, tgt_ns):
    pred_dsmat = pred_dsmat.astype(jnp.float32)
    gt_perm = gt_perm.astype(jnp.float32)

    idx16, lane = pl.pallas_call(
        _gt_idx_body,
        grid=(_B, _NBLK),
        in_specs=[
            pl.BlockSpec(memory_space=pltpu.SMEM),
            pl.BlockSpec((1, _R, _N2), lambda b, r: (b, r, 0)),
        ],
        out_specs=[
            pl.BlockSpec((_R, 1), lambda b, r: (b * _NBLK + r, 0)),
            pl.BlockSpec((_R, 1), lambda b, r: (b * _NBLK + r, 0)),
        ],
        out_shape=[
            jax.ShapeDtypeStruct((_NROWS, 1), jnp.int32),
            jax.ShapeDtypeStruct((_NROWS, 1), jnp.int32),
        ],
    )(tgt_ns, gt_perm)

    m1, m2 = pl.pallas_call(
        _top2_body,
        grid=(_B, _NBLK),
        in_specs=[
            pl.BlockSpec(memory_space=pltpu.SMEM),
            pl.BlockSpec((1, _R, _N2), lambda b, r: (b, r, 0)),
        ],
        out_specs=[
            pl.BlockSpec((_R, 1), lambda b, r: (b * _NBLK + r, 0)),
            pl.BlockSpec((_R, 1), lambda b, r: (b * _NBLK + r, 0)),
        ],
        out_shape=[
            jax.ShapeDtypeStruct((_NROWS, 1), jnp.float32),
            jax.ShapeDtypeStruct((_NROWS, 1), jnp.float32),
        ],
    )(tgt_ns, pred_dsmat)

    table = pred_dsmat.reshape(_B * _N1 * (_N2 // _GW), _GW)
    pag = _sc_gather(table, idx16.reshape(_NROWS), lane.reshape(_NROWS))

    out = pl.pallas_call(
        _combine_body,
        in_specs=[
            pl.BlockSpec(memory_space=pltpu.SMEM),
            pl.BlockSpec((_B, _N1), lambda: (0, 0)),
            pl.BlockSpec((_B, _N1), lambda: (0, 0)),
            pl.BlockSpec((_B, _N1), lambda: (0, 0)),
        ],
        out_specs=pl.BlockSpec(memory_space=pltpu.SMEM),
        out_shape=jax.ShapeDtypeStruct((1, 1), jnp.float32),
    )(
        src_ns,
        m1.reshape(_B, _N1),
        m2.reshape(_B, _N1),
        pag.reshape(_B, _N1),
    )
    return out[0, 0]


# speedup vs baseline: 2.7305x; 2.7305x over previous
import jax
import jax.numpy as jnp
from jax import lax
from jax.experimental import pallas as pl
from jax.experimental.pallas import tpu as pltpu

_B, _N1, _N2 = 16, 1024, 1024
_NROWS = _B * _N1


def _gt_body(tgt_ref, gt_ref, gidx_ref):
    b = pl.program_id(0)
    nt = tgt_ref[b]
    gx = gt_ref[0]
    col = lax.broadcasted_iota(jnp.int32, (_N1, _N2), 1)
    neg = jnp.float32(-jnp.inf)
    mg = jnp.where(col < nt, gx, neg)
    g1 = jnp.max(mg, axis=1, keepdims=True)
    gidx_ref[...] = jnp.min(jnp.where(mg == g1, col, _N2), axis=1, keepdims=True)


def _main_body(tgt_ref, src_ref, pred_ref, gidx_ref, out_ref, acc_ref):
    b = pl.program_id(0)
    nt = tgt_ref[b]
    ns = src_ref[b]
    px = pred_ref[0]
    gidx = gidx_ref[...]
    col = lax.broadcasted_iota(jnp.int32, (_N1, _N2), 1)
    neg = jnp.float32(-jnp.inf)

    mp = jnp.where(col < nt, px, neg)
    h = jnp.maximum(mp[:, :512], mp[:, 512:])
    l = jnp.minimum(mp[:, :512], mp[:, 512:])
    for w in (256, 128):
        h1, h2 = h[:, :w], h[:, w:]
        l = jnp.maximum(jnp.minimum(h1, h2), jnp.maximum(l[:, :w], l[:, w:]))
        h = jnp.maximum(h1, h2)
    m1 = jnp.max(h, axis=1, keepdims=True)
    is_m1 = h == m1
    m2h_strict = jnp.max(jnp.where(is_m1, neg, h), axis=1, keepdims=True)
    dup = jnp.sum(jnp.where(is_m1, 1, 0), axis=1, keepdims=True) > 1
    m2h = jnp.where(dup, m1, m2h_strict)
    m2 = jnp.maximum(m2h, jnp.max(l, axis=1, keepdims=True))

    grp = lax.shift_right_logical(gidx, 7)
    strip = px[:, :128]
    for g in range(1, 8):
        strip = jnp.where(grp == g, px[:, g * 128 : (g + 1) * 128], strip)
    lane = lax.bitwise_and(gidx, 127)
    col128 = lax.broadcasted_iota(jnp.int32, (_N1, 128), 1)
    pag = jnp.max(jnp.where(col128 == lane, strip, neg), axis=1, keepdims=True)

    sel = jnp.where(pag == m1, m2, m1)
    contrib = jnp.log(sel) - jnp.log(pag)
    row = lax.broadcasted_iota(jnp.int32, (_N1, 1), 0)
    partial = jnp.sum(jnp.where(row < ns, contrib, 0.0))

    acc_ref[0] = jnp.where(b == 0, 0.0, acc_ref[0]) + partial

    @pl.when(b == _B - 1)
    def _():
        n_sum = lax.fori_loop(
            0, _B, lambda i, s: s + src_ref[i].astype(jnp.float32), jnp.float32(0.0)
        )
        out_ref[0, 0] = acc_ref[0] / n_sum


def kernel(pred_dsmat, gt_perm, src_ns, tgt_ns):
    pred_dsmat = pred_dsmat.astype(jnp.float32)
    gt_perm = gt_perm.astype(jnp.float32)

    gidx = pl.pallas_call(
        _gt_body,
        grid=(_B,),
        in_specs=[
            pl.BlockSpec(memory_space=pltpu.SMEM),
            pl.BlockSpec((1, _N1, _N2), lambda b: (b, 0, 0)),
        ],
        out_specs=pl.BlockSpec((_N1, 1), lambda b: (b, 0)),
        out_shape=jax.ShapeDtypeStruct((_NROWS, 1), jnp.int32),
    )(tgt_ns, gt_perm)

    out = pl.pallas_call(
        _main_body,
        grid=(_B,),
        in_specs=[
            pl.BlockSpec(memory_space=pltpu.SMEM),
            pl.BlockSpec(memory_space=pltpu.SMEM),
            pl.BlockSpec((1, _N1, _N2), lambda b: (b, 0, 0)),
            pl.BlockSpec((_N1, 1), lambda b: (b, 0)),
        ],
        out_specs=pl.BlockSpec(memory_space=pltpu.SMEM),
        out_shape=jax.ShapeDtypeStruct((1, 1), jnp.float32),
        scratch_shapes=[pltpu.SMEM((1,), jnp.float32)],
    )(tgt_ns, src_ns, pred_dsmat, gidx)
    return out[0, 0]


# speedup vs baseline: 4.3108x; 1.5787x over previous
import jax
import jax.numpy as jnp
from jax import lax
from jax.experimental import pallas as pl
from jax.experimental.pallas import tpu as pltpu

_B, _N1, _N2 = 16, 1024, 1024


def _cw_body(tgt_ref, src_ref, pred_ref, gt_ref, out_ref, acc_ref):
    b = pl.program_id(0)
    nt = tgt_ref[b]
    ns = src_ref[b]
    px = pred_ref[0]
    gx = gt_ref[0]
    neg = jnp.float32(-jnp.inf)

    col1 = lax.broadcasted_iota(jnp.int32, (1, _N2), 1)
    maskrow = jnp.where(col1 < nt, 0.0, neg)

    mg = gx + maskrow
    g1 = jnp.max(mg, axis=1, keepdims=True)
    col = lax.broadcasted_iota(jnp.int32, (_N1, _N2), 1)
    gidx = jnp.min(jnp.where(mg == g1, col, _N2), axis=1, keepdims=True)

    mp = px + maskrow
    h = jnp.maximum(mp[:, :512], mp[:, 512:])
    l = jnp.minimum(mp[:, :512], mp[:, 512:])
    for w in (256, 128):
        h1, h2 = h[:, :w], h[:, w:]
        l = jnp.maximum(jnp.minimum(h1, h2), jnp.maximum(l[:, :w], l[:, w:]))
        h = jnp.maximum(h1, h2)
    m1 = jnp.max(h, axis=1, keepdims=True)
    is_m1 = h == m1
    z = jnp.maximum(jnp.where(is_m1, neg, h), l)
    m2_strict = jnp.max(z, axis=1, keepdims=True)
    dup = jnp.sum(is_m1, axis=1, keepdims=True) > 1
    m2 = jnp.where(dup, m1, m2_strict)

    grp = lax.shift_right_logical(gidx, 7)
    strip = px[:, :128]
    for g in range(1, 8):
        strip = jnp.where(grp == g, px[:, g * 128 : (g + 1) * 128], strip)
    lane = lax.bitwise_and(gidx, 127)
    col128 = lax.broadcasted_iota(jnp.int32, (_N1, 128), 1)
    pag = jnp.max(jnp.where(col128 == lane, strip, neg), axis=1, keepdims=True)

    sel = jnp.where(pag == m1, m2, m1)
    contrib = jnp.log(sel) - jnp.log(pag)
    row = lax.broadcasted_iota(jnp.int32, (_N1, 1), 0)
    partial = jnp.sum(jnp.where(row < ns, contrib, 0.0))

    acc_ref[0] = jnp.where(b == 0, 0.0, acc_ref[0]) + partial

    @pl.when(b == _B - 1)
    def _():
        n_sum = lax.fori_loop(
            0, _B, lambda i, s: s + src_ref[i].astype(jnp.float32), jnp.float32(0.0)
        )
        out_ref[0, 0] = acc_ref[0] / n_sum


def kernel(pred_dsmat, gt_perm, src_ns, tgt_ns):
    pred_dsmat = pred_dsmat.astype(jnp.float32)
    gt_perm = gt_perm.astype(jnp.float32)

    out = pl.pallas_call(
        _cw_body,
        grid=(_B,),
        in_specs=[
            pl.BlockSpec(memory_space=pltpu.SMEM),
            pl.BlockSpec(memory_space=pltpu.SMEM),
            pl.BlockSpec((1, _N1, _N2), lambda b: (b, 0, 0)),
            pl.BlockSpec((1, _N1, _N2), lambda b: (b, 0, 0)),
        ],
        out_specs=pl.BlockSpec(memory_space=pltpu.SMEM),
        out_shape=jax.ShapeDtypeStruct((1, 1), jnp.float32),
        scratch_shapes=[pltpu.SMEM((1,), jnp.float32)],
    )(tgt_ns, src_ns, pred_dsmat, gt_perm)
    return out[0, 0]
